# baseline (speedup 1.0000x reference)
import functools

import jax
import jax.numpy as jnp
from jax import lax
from jax.experimental import pallas as pl
from jax.experimental.pallas import tpu as pltpu
from jax.experimental.pallas import tpu_sc as plsc

EMB = 64
HID = 64
OUT = 32
NC = 2
NS = 16
NW = NC * NS


def _sc_gather_quads(emb4, idx_flat):
    b = idx_flat.shape[0]
    b_per_w = b // NW
    mesh = plsc.VectorSubcoreMesh(core_axis_name="c", subcore_axis_name="s")

    @functools.partial(
        pl.kernel,
        mesh=mesh,
        out_type=jax.ShapeDtypeStruct((b, 128), jnp.int32),
        scratch_types=[
            pltpu.VMEM((512,), jnp.int32),
            pltpu.VMEM((512, 128), jnp.int32),
            pltpu.SemaphoreType.DMA,
        ],
    )
    def gather_kernel(table_hbm, idx_hbm, out_hbm, idx_v, rows_v, sem):
        wid = lax.axis_index("s") * NC + lax.axis_index("c")
        base = wid * b_per_w

        @pl.loop(0, b_per_w, step=512)
        def _(off):
            pltpu.sync_copy(idx_hbm.at[pl.ds(base + off, 512)], idx_v)
            pltpu.async_copy(table_hbm.at[idx_v], rows_v, sem).wait()
            pltpu.sync_copy(rows_v, out_hbm.at[pl.ds(base + off, 512)])

    return gather_kernel(emb4, idx_flat)


def _mlp(g, q0, q1, w1a_ev, w1a_od, w1b_ev, w1b_od, b1, w2T, b2, w3T, b3):
    n = g.shape[0] // 2
    blk = 2048
    nb = n // blk
    def _quarter(v, q):
        half = jnp.where(q >= 2, v[:, 64:], v[:, :64])
        return jnp.where(q % 2 != 0, half[:, 32:], half[:, :32])

    def body(g0_ref, g1_ref, q0_ref, q1_ref, w1a_ev_ref, w1a_od_ref,
             w1b_ev_ref, w1b_od_ref, b1_ref, w2_ref, b2_ref, w3_ref,
             b3_ref, o_ref):
        g0 = g0_ref[...]
        g1 = g1_ref[...]
        q0 = q0_ref[...]
        q1 = q1_ref[...]
        mask_hi = jnp.full(g0.shape, -65536, jnp.int32)
        f32 = lambda v: jax.lax.bitcast_convert_type(v, jnp.float32)
        x1_ev = _quarter(f32(g0 << 16), q0)
        x1_od = _quarter(f32(g0 & mask_hi), q0)
        x2_ev = _quarter(f32(g1 << 16), q1)
        x2_od = _quarter(f32(g1 & mask_hi), q1)
        dot = lambda a_, b_: jnp.dot(a_, b_, preferred_element_type=jnp.float32)
        a = dot(x1_ev, w1a_ev_ref[...]) + dot(x1_od, w1a_od_ref[...])
        a = a + dot(x2_ev, w1b_ev_ref[...]) + dot(x2_od, w1b_od_ref[...])
        a = jnp.maximum(a + b1_ref[...], 0.0)
        a = jnp.maximum(dot(a, w2_ref[...]) + b2_ref[...], 0.0)
        o_ref[...] = dot(a, w3_ref[...]) + b3_ref[...]

    full = lambda shape: pl.BlockSpec(shape, lambda i: (0, 0))
    return pl.pallas_call(
        body,
        grid=(nb,),
        in_specs=[
            pl.BlockSpec((blk, 128), lambda i: (i, 0)),
            pl.BlockSpec((blk, 128), lambda i: (i + nb, 0)),
            pl.BlockSpec((blk, 1), lambda i: (i, 0)),
            pl.BlockSpec((blk, 1), lambda i: (i, 0)),
            full((EMB // 2, HID)),
            full((EMB // 2, HID)),
            full((EMB // 2, HID)),
            full((EMB // 2, HID)),
            full((1, HID)),
            full((HID, HID)),
            full((1, HID)),
            full((HID, OUT)),
            full((1, OUT)),
        ],
        out_specs=pl.BlockSpec((blk, OUT), lambda i: (i, 0)),
        out_shape=jax.ShapeDtypeStruct((n, OUT), jnp.float32),
    )(g, g, q0, q1, w1a_ev, w1a_od, w1b_ev, w1b_od, b1, w2T, b2, w3T, b3)


def kernel(x, emb, w1, b1, w2, b2, w3, b3):
    xi = x.astype(jnp.int32)
    idx_flat = xi.T.reshape(-1)
    quad_idx = idx_flat // 4
    quarter = idx_flat % 4
    n = xi.shape[0]
    emb4 = jax.lax.bitcast_convert_type(
        emb.astype(jnp.bfloat16).reshape(emb.shape[0] // 4, 128, 2),
        jnp.int32,
    )
    g = _sc_gather_quads(emb4, quad_idx)
    w1aT = w1[:, :EMB].T
    w1bT = w1[:, EMB:].T
    return _mlp(
        g,
        quarter[:n].reshape(n, 1),
        quarter[n:].reshape(n, 1),
        w1aT[0::2],
        w1aT[1::2],
        w1bT[0::2],
        w1bT[1::2],
        b1.reshape(1, HID),
        w2.T,
        b2.reshape(1, HID),
        w3.T,
        b3.reshape(1, OUT),
    )


# speedup vs baseline: 52.1310x; 52.1310x over previous
import functools

import jax
import jax.numpy as jnp
from jax import lax
from jax.experimental import pallas as pl
from jax.experimental.pallas import tpu as pltpu
from jax.experimental.pallas import tpu_sc as plsc

EMB = 64
HID = 64
OUT = 32
NC = 2
NS = 16
NW = NC * NS
LANES = 16
CHUNK = 512


def _sc_gather_pairs(embbf, idx_even):
    m = idx_even.shape[0]
    per_w = m // NW
    mesh = plsc.VectorSubcoreMesh(core_axis_name="c", subcore_axis_name="s")

    @functools.partial(
        pl.kernel,
        mesh=mesh,
        out_type=jax.ShapeDtypeStruct((2 * m, EMB), jnp.bfloat16),
        scratch_types=[
            pltpu.VMEM((per_w,), jnp.int32),
            pltpu.VMEM((2 * CHUNK, EMB), jnp.bfloat16),
            pltpu.SemaphoreType.DMA,
        ],
    )
    def gather_kernel(table_hbm, idx_hbm, out_hbm, i_v, rows_v, sem):
        wid = lax.axis_index("s") * NC + lax.axis_index("c")
        base = wid * per_w
        pltpu.sync_copy(idx_hbm.at[pl.ds(base, per_w)], i_v)

        @pl.loop(0, per_w, step=CHUNK)
        def _(off):
            @pl.loop(0, CHUNK, step=LANES)
            def _(j0):
                vec = i_v[pl.ds(off + j0, LANES)]
                for t in range(LANES):
                    a = pl.multiple_of(vec[t], 2)
                    pltpu.async_copy(
                        table_hbm.at[pl.ds(a, 2)],
                        rows_v.at[pl.ds(2 * (j0 + t), 2)],
                        sem,
                    )

            pltpu.make_async_copy(
                table_hbm.at[pl.ds(0, 2 * CHUNK)], rows_v, sem
            ).wait()
            pltpu.sync_copy(
                rows_v,
                out_hbm.at[pl.ds(pl.multiple_of(2 * (base + off), 1024),
                                 2 * CHUNK)],
            )

    return gather_kernel(embbf, idx_even)


def _mlp(g, p0, p1, w1aT, w1bT, b1, w2T, b2, w3T, b3):
    n = g.shape[0] // 4
    blk = 2048
    nb = n // blk

    def _pick(r, p):
        r2 = r.astype(jnp.float32).reshape(r.shape[0] // 2, 2, EMB)
        return jnp.where(p != 0, r2[:, 1, :], r2[:, 0, :])

    def body(g0_ref, g1_ref, p0_ref, p1_ref, w1a_ref, w1b_ref, b1_ref,
             w2_ref, b2_ref, w3_ref, b3_ref, o_ref):
        x1 = _pick(g0_ref[...], p0_ref[...])
        x2 = _pick(g1_ref[...], p1_ref[...])
        dot = lambda a_, b_: jnp.dot(a_, b_, preferred_element_type=jnp.float32)
        a = dot(x1, w1a_ref[...]) + dot(x2, w1b_ref[...])
        a = jnp.maximum(a + b1_ref[...], 0.0)
        a = jnp.maximum(dot(a, w2_ref[...]) + b2_ref[...], 0.0)
        o_ref[...] = dot(a, w3_ref[...]) + b3_ref[...]

    full = lambda shape: pl.BlockSpec(shape, lambda i: (0, 0))
    return pl.pallas_call(
        body,
        grid=(nb,),
        in_specs=[
            pl.BlockSpec((2 * blk, EMB), lambda i: (i, 0)),
            pl.BlockSpec((2 * blk, EMB), lambda i: (i + nb, 0)),
            pl.BlockSpec((blk, 1), lambda i: (i, 0)),
            pl.BlockSpec((blk, 1), lambda i: (i, 0)),
            full((EMB, HID)),
            full((EMB, HID)),
            full((1, HID)),
            full((HID, HID)),
            full((1, HID)),
            full((HID, OUT)),
            full((1, OUT)),
        ],
        out_specs=pl.BlockSpec((blk, OUT), lambda i: (i, 0)),
        out_shape=jax.ShapeDtypeStruct((n, OUT), jnp.float32),
    )(g, g, p0, p1, w1aT, w1bT, b1, w2T, b2, w3T, b3)


def kernel(x, emb, w1, b1, w2, b2, w3, b3):
    xi = x.astype(jnp.int32)
    idx_flat = xi.T.reshape(-1)
    idx_even = (idx_flat // 2) * 2
    parity = idx_flat % 2
    n = xi.shape[0]
    embbf = emb.astype(jnp.bfloat16)
    g = _sc_gather_pairs(embbf, idx_even)
    return _mlp(
        g,
        parity[:n].reshape(n, 1),
        parity[n:].reshape(n, 1),
        w1[:, :EMB].T,
        w1[:, EMB:].T,
        b1.reshape(1, HID),
        w2.T,
        b2.reshape(1, HID),
        w3.T,
        b3.reshape(1, OUT),
    )


# speedup vs baseline: 55.8211x; 1.0708x over previous
import functools

import jax
import jax.numpy as jnp
from jax import lax
from jax.experimental import pallas as pl
from jax.experimental.pallas import tpu as pltpu
from jax.experimental.pallas import tpu_sc as plsc

EMB = 64
HID = 64
OUT = 32
NC = 2
NS = 16
NW = NC * NS
LANES = 16
CHUNK = 512


def _sc_gather_pairs(embbf, idx_even):
    m = idx_even.shape[0]
    per_w = m // NW
    mesh = plsc.VectorSubcoreMesh(core_axis_name="c", subcore_axis_name="s")

    @functools.partial(
        pl.kernel,
        mesh=mesh,
        out_type=jax.ShapeDtypeStruct((2 * m, EMB), jnp.bfloat16),
        scratch_types=[
            pltpu.VMEM((per_w,), jnp.int32),
            pltpu.VMEM((2 * CHUNK, EMB), jnp.bfloat16),
            pltpu.SemaphoreType.DMA,
        ],
    )
    def gather_kernel(table_hbm, idx_hbm, out_hbm, i_v, rows_v, sem):
        wid = lax.axis_index("s") * NC + lax.axis_index("c")
        base = wid * per_w
        pltpu.sync_copy(idx_hbm.at[pl.ds(base, per_w)], i_v)

        @pl.loop(0, per_w, step=CHUNK)
        def _(off):
            @pl.loop(0, CHUNK, step=LANES)
            def _(j0):
                vec = i_v[pl.ds(off + j0, LANES)]
                for t in range(LANES):
                    a = pl.multiple_of(vec[t], 2)
                    pltpu.async_copy(
                        table_hbm.at[pl.ds(a, 2)],
                        rows_v.at[pl.ds(2 * (j0 + t), 2)],
                        sem,
                    )

            pltpu.make_async_copy(
                table_hbm.at[pl.ds(0, 2 * CHUNK)], rows_v, sem
            ).wait()
            pltpu.sync_copy(
                rows_v,
                out_hbm.at[pl.ds(pl.multiple_of(2 * (base + off), 1024),
                                 2 * CHUNK)],
            )

    return gather_kernel(embbf, idx_even)


def _mlp(g, p0, p1, w1aT, w1bT, b1, w2T, b2, w3T, b3):
    n = g.shape[0] // 4
    blk = 2048
    nb = n // blk

    def _pick(r, p):
        r2 = r.astype(jnp.float32).reshape(r.shape[0] // 2, 2, EMB)
        return jnp.where(p != 0, r2[:, 1, :], r2[:, 0, :])

    def body(g0_ref, g1_ref, p0_ref, p1_ref, w1a_ref, w1b_ref, b1_ref,
             w2_ref, b2_ref, w3_ref, b3_ref, o_ref):
        x1 = _pick(g0_ref[...], p0_ref[...])
        x2 = _pick(g1_ref[...], p1_ref[...])
        dot = lambda a_, b_: jnp.dot(a_, b_, preferred_element_type=jnp.float32)
        a = dot(x1, w1a_ref[...]) + dot(x2, w1b_ref[...])
        a = jnp.maximum(a + b1_ref[...], 0.0)
        a = jnp.maximum(dot(a, w2_ref[...]) + b2_ref[...], 0.0)
        o_ref[...] = dot(a, w3_ref[...]) + b3_ref[...]

    full = lambda shape: pl.BlockSpec(shape, lambda i: (0, 0))
    return pl.pallas_call(
        body,
        grid=(nb,),
        in_specs=[
            pl.BlockSpec((2 * blk, EMB), lambda i: (i, 0)),
            pl.BlockSpec((2 * blk, EMB), lambda i: (i + nb, 0)),
            pl.BlockSpec((blk, 1), lambda i: (i, 0)),
            pl.BlockSpec((blk, 1), lambda i: (i, 0)),
            full((EMB, HID)),
            full((EMB, HID)),
            full((1, HID)),
            full((HID, HID)),
            full((1, HID)),
            full((HID, OUT)),
            full((1, OUT)),
        ],
        out_specs=pl.BlockSpec((blk, OUT), lambda i: (i, 0)),
        out_shape=jax.ShapeDtypeStruct((n, OUT), jnp.float32),
    )(g, g, p0, p1, w1aT, w1bT, b1, w2T, b2, w3T, b3)


def kernel(x, emb, w1, b1, w2, b2, w3, b3):
    xi = x.astype(jnp.int32)
    idx_flat = xi.T.reshape(-1)
    idx_even = (idx_flat // 2) * 2
    parity = idx_flat % 2
    n = xi.shape[0]
    embbf = emb.astype(jnp.bfloat16)
    g = _sc_gather_pairs(embbf, idx_even)
    return g
    return _mlp(
        g,
        parity[:n].reshape(n, 1),
        parity[n:].reshape(n, 1),
        w1[:, :EMB].T,
        w1[:, EMB:].T,
        b1.reshape(1, HID),
        w2.T,
        b2.reshape(1, HID),
        w3.T,
        b3.reshape(1, OUT),
    )


# speedup vs baseline: 56.9286x; 1.0198x over previous
import functools

import jax
import jax.numpy as jnp
from jax import lax
from jax.experimental import pallas as pl
from jax.experimental.pallas import tpu as pltpu
from jax.experimental.pallas import tpu_sc as plsc

EMB = 64
HID = 64
OUT = 32
NC = 2
NS = 16
NW = NC * NS
LANES = 16


def _sc_gather(emb, idx_flat):
    m = idx_flat.shape[0]
    per_w = m // NW
    mesh = plsc.VectorSubcoreMesh(core_axis_name="c", subcore_axis_name="s")

    @functools.partial(
        pl.kernel,
        mesh=mesh,
        out_type=jax.ShapeDtypeStruct((m, EMB), jnp.float32),
        scratch_types=[
            pltpu.VMEM((per_w,), jnp.int32),
            pltpu.VMEM((512, EMB), jnp.float32),
            pltpu.SemaphoreType.DMA,
        ],
    )
    def gather_kernel(table_hbm, idx_hbm, out_hbm, i_v, rows_v, sem):
        wid = lax.axis_index("s") * NC + lax.axis_index("c")
        base = wid * per_w
        pltpu.sync_copy(idx_hbm.at[pl.ds(base, per_w)], i_v)

        @pl.loop(0, per_w, step=512)
        def _(off):
            @pl.loop(0, 512, step=LANES)
            def _(j0):
                vec = i_v[pl.ds(off + j0, LANES)]
                for t in range(LANES):
                    a = vec[t]
                    pltpu.async_copy(
                        table_hbm.at[pl.ds(a, 1)],
                        rows_v.at[pl.ds(j0 + t, 1)],
                        sem,
                    )

            pltpu.make_async_copy(
                table_hbm.at[pl.ds(0, 512)], rows_v, sem
            ).wait()
            pltpu.sync_copy(rows_v, out_hbm.at[pl.ds(base + off, 512)])

    return gather_kernel(emb, idx_flat)


def _mlp(g, w1aT, w1bT, b1, w2T, b2, w3T, b3):
    n = g.shape[0] // 2
    blk = 2048
    nb = n // blk

    def body(x1_ref, x2_ref, w1a_ref, w1b_ref, b1_ref, w2_ref, b2_ref,
             w3_ref, b3_ref, o_ref):
        a = jnp.dot(x1_ref[...], w1a_ref[...], preferred_element_type=jnp.float32)
        a = a + jnp.dot(x2_ref[...], w1b_ref[...], preferred_element_type=jnp.float32)
        a = jnp.maximum(a + b1_ref[...], 0.0)
        a = jnp.dot(a, w2_ref[...], preferred_element_type=jnp.float32) + b2_ref[...]
        a = jnp.maximum(a, 0.0)
        o_ref[...] = jnp.dot(a, w3_ref[...], preferred_element_type=jnp.float32) + b3_ref[...]

    full = lambda shape: pl.BlockSpec(shape, lambda i: (0, 0))
    return pl.pallas_call(
        body,
        grid=(nb,),
        in_specs=[
            pl.BlockSpec((blk, EMB), lambda i: (i, 0)),
            pl.BlockSpec((blk, EMB), lambda i: (i + nb, 0)),
            full((EMB, HID)),
            full((EMB, HID)),
            full((1, HID)),
            full((HID, HID)),
            full((1, HID)),
            full((HID, OUT)),
            full((1, OUT)),
        ],
        out_specs=pl.BlockSpec((blk, OUT), lambda i: (i, 0)),
        out_shape=jax.ShapeDtypeStruct((n, OUT), jnp.float32),
    )(g, g, w1aT, w1bT, b1, w2T, b2, w3T, b3)


def kernel(x, emb, w1, b1, w2, b2, w3, b3):
    xi = x.astype(jnp.int32)
    idx_flat = xi.T.reshape(-1)
    g = _sc_gather(emb, idx_flat)
    return _mlp(
        g,
        w1[:, :EMB].T,
        w1[:, EMB:].T,
        b1.reshape(1, HID),
        w2.T,
        b2.reshape(1, HID),
        w3.T,
        b3.reshape(1, OUT),
    )
